# baseline (speedup 1.0000x reference)
import jax
import jax.numpy as jnp
from jax import lax
from jax.experimental import pallas as pl
from jax.experimental.pallas import tpu as pltpu
from jax.experimental.pallas import tpu_sc as plsc

_N = 100000
_NW = 32
_CHUNK = 3136
_ITERS = _CHUNK // 16


def _body(x0_hbm, w_hbm, o0_hbm, o1_hbm, o2_hbm,
          x0_v, w_v, o0_v, o1_v, o2_v):
    nc = 2
    wid = lax.axis_index("s") * nc + lax.axis_index("c")
    base = jnp.where(wid == _NW - 1, _N - _CHUNK, wid * _CHUNK)

    pltpu.sync_copy(w_hbm, w_v)
    pltpu.sync_copy(x0_hbm.at[pl.ds(base, _CHUNK)], x0_v)

    j0 = jnp.zeros((16,), jnp.int32)
    j1 = jnp.ones((16,), jnp.int32)
    j2 = jnp.full((16,), 2, jnp.int32)

    @plsc.parallel_loop(0, _ITERS, unroll=14)
    def _step(i):
        s = pl.ds(i * 16, 16)
        idx = x0_v[s]
        o0_v[s] = plsc.load_gather(w_v, [idx, j0])
        o1_v[s] = plsc.load_gather(w_v, [idx, j1])
        o2_v[s] = plsc.load_gather(w_v, [idx, j2])

    pltpu.sync_copy(o0_v, o0_hbm.at[pl.ds(base, _CHUNK)])
    pltpu.sync_copy(o1_v, o1_hbm.at[pl.ds(base, _CHUNK)])
    pltpu.sync_copy(o2_v, o2_hbm.at[pl.ds(base, _CHUNK)])


_plane = jax.ShapeDtypeStruct((_N,), jnp.float32)
_sc_call = pl.kernel(
    _body,
    out_type=(_plane, _plane, _plane),
    mesh=plsc.VectorSubcoreMesh(core_axis_name="c", subcore_axis_name="s"),
    scratch_types=[
        pltpu.VMEM((_CHUNK,), jnp.int32),
        pltpu.VMEM((3, 3), jnp.float32),
        pltpu.VMEM((_CHUNK,), jnp.float32),
        pltpu.VMEM((_CHUNK,), jnp.float32),
        pltpu.VMEM((_CHUNK,), jnp.float32),
    ],
    compiler_params=pltpu.CompilerParams(needs_layout_passes=False),
)


def kernel(x, W):
    x0 = x[:, 0]
    o0, o1, o2 = _sc_call(x0, W)
    o3 = x[:, 1].astype(jnp.float32)
    return jnp.concatenate(
        (o0[:, None], o1[:, None], o2[:, None], o3[:, None]), axis=1
    )


# speedup vs baseline: 1.1871x; 1.1871x over previous
import jax
import jax.numpy as jnp
from jax import lax
from jax.experimental import pallas as pl
from jax.experimental.pallas import tpu as pltpu
from jax.experimental.pallas import tpu_sc as plsc

_N = 100000
_NW = 16
_CHUNK = 6272
_ITERS = _CHUNK // 16


def _body(x0_hbm, w_hbm, o_hbm,
          x0_v, w_v, o0_v, o1_v, o2_v):
    wid = lax.axis_index("s")
    base = jnp.where(wid == _NW - 1, _N - _CHUNK, wid * _CHUNK)

    pltpu.sync_copy(w_hbm, w_v)
    pltpu.sync_copy(x0_hbm.at[pl.ds(base, _CHUNK)], x0_v)

    @plsc.parallel_loop(0, _ITERS, unroll=14)
    def _step(i):
        s = pl.ds(i * 16, 16)
        wbase = x0_v[s] * 3
        o0_v[s] = plsc.load_gather(w_v, [wbase])
        o1_v[s] = plsc.load_gather(w_v, [wbase + 1])
        o2_v[s] = plsc.load_gather(w_v, [wbase + 2])

    pltpu.sync_copy(o0_v, o_hbm.at[pl.ds(base, _CHUNK)])
    pltpu.sync_copy(o1_v, o_hbm.at[pl.ds(_N + base, _CHUNK)])
    pltpu.sync_copy(o2_v, o_hbm.at[pl.ds(2 * _N + base, _CHUNK)])


_sc_call = pl.kernel(
    _body,
    out_type=jax.ShapeDtypeStruct((3 * _N,), jnp.float32),
    mesh=plsc.VectorSubcoreMesh(core_axis_name="c", subcore_axis_name="s", num_cores=1),
    scratch_types=[
        pltpu.VMEM((_CHUNK,), jnp.int32),
        pltpu.VMEM((16,), jnp.float32),
        pltpu.VMEM((_CHUNK,), jnp.float32),
        pltpu.VMEM((_CHUNK,), jnp.float32),
        pltpu.VMEM((_CHUNK,), jnp.float32),
    ],
    compiler_params=pltpu.CompilerParams(needs_layout_passes=False),
)


def kernel(x, W):
    x0 = x[:, 0]
    w16 = jnp.pad(W.reshape(-1), (0, 7))
    planes = _sc_call(x0, w16)
    o0 = planes[:_N]
    o1 = planes[_N:2 * _N]
    o2 = planes[2 * _N:]
    o3 = x[:, 1].astype(jnp.float32)
    return jnp.concatenate(
        (o0[:, None], o1[:, None], o2[:, None], o3[:, None]), axis=1
    )
